# baseline (speedup 1.0000x reference)
import functools

import jax
import jax.numpy as jnp
from jax import lax
from jax.experimental import pallas as pl
from jax.experimental.pallas import tpu as pltpu
from jax.experimental.pallas import tpu_sc as plsc

N = 10000
D = 128
E = 320000
NUM_ITERS = 5

NC = 2
NS = 16
NW = NC * NS
EPW = E // NW
CHUNK = 400
NCHUNKS = EPW // CHUNK
NPAD = 10240
RPT = NPAD // NS
DH = D // NC
EPAD = 332800
EPS = EPAD // NS
CHD = 320
NCHD = EPS // CHD

_MESH = dict(core_axis_name="c", subcore_axis_name="s")


def _tc_normalize(x):
    def body(x_ref, xn_ref):
        xv = x_ref[...]
        sq = jnp.sum(xv * xv, axis=1, keepdims=True)
        xn_ref[...] = xv * lax.rsqrt(jnp.maximum(sq, 1e-12))

    return pl.pallas_call(
        body, out_shape=jax.ShapeDtypeStruct((N, D), jnp.float32)
    )(x)


def _sc_edge_weights(xn, rowi, coli, ev, scalin):
    @functools.partial(
        pl.kernel,
        out_type=(
            jax.ShapeDtypeStruct((E,), jnp.float32),
            jax.ShapeDtypeStruct((NC, NPAD), jnp.float32),
        ),
        mesh=plsc.VectorSubcoreMesh(**_MESH),
        compiler_params=pltpu.CompilerParams(needs_layout_passes=False),
        scratch_types=[
            pltpu.VMEM((CHUNK,), jnp.int32),
            pltpu.VMEM((CHUNK,), jnp.int32),
            pltpu.VMEM((CHUNK,), jnp.float32),
            pltpu.VMEM((CHUNK,), jnp.float32),
            pltpu.VMEM((CHUNK, D), jnp.float32),
            pltpu.VMEM((CHUNK, D), jnp.float32),
            pltpu.VMEM((RPT,), jnp.float32),
            pltpu.VMEM((16,), jnp.float32),
            pltpu.VMEM_SHARED((NPAD,), jnp.float32),
            pltpu.SemaphoreType.DMA,
            pltpu.SemaphoreType.DMA,
        ],
    )
    def k(xn_h, row_h, col_h, ev_h, scal_h, w_h, np_h,
          ri, ci, evv, wv, bufA, bufB, zbuf, scal_v, nshared, sem, sem2):
        cid = lax.axis_index("c")
        sid = lax.axis_index("s")
        wid = sid * NC + cid
        base = wid * EPW

        def z16(g, _):
            zbuf[pl.ds(g * 16, 16)] = jnp.zeros((16,), jnp.float32)
            return 0

        lax.fori_loop(0, RPT // 16, z16, 0)
        pltpu.sync_copy(zbuf, nshared.at[pl.ds(sid * RPT, RPT)])
        plsc.subcore_barrier()

        pltpu.sync_copy(scal_h, scal_v)
        sv = jnp.exp(scal_v[...] * -2.0)
        scale = 0.25 * sv[2]

        def chunk(g, _):
            off = base + g * CHUNK
            pltpu.sync_copy(row_h.at[pl.ds(off, CHUNK)], ri)
            pltpu.sync_copy(col_h.at[pl.ds(off, CHUNK)], ci)
            pltpu.sync_copy(ev_h.at[pl.ds(off, CHUNK)], evv)
            da = pltpu.async_copy(xn_h.at[ri], bufA, sem)
            db = pltpu.async_copy(xn_h.at[ci], bufB, sem2)
            da.wait()
            db.wait()

            lane = lax.iota(jnp.int32, 16)

            def group(gg, _):
                sl = pl.ds(gg * 16, 16)
                dvec = jnp.zeros((16,), jnp.float32)
                for k in range(16):
                    e = gg * 16 + k
                    p = [bufA[e, pl.ds(s * 16, 16)] * bufB[e, pl.ds(s * 16, 16)]
                         for s in range(8)]
                    acc = (((p[0] + p[1]) + (p[2] + p[3]))
                           + ((p[4] + p[5]) + (p[6] + p[7])))
                    dsum = jnp.sum(acc)
                    dvec = jnp.where(lane == k, dsum, dvec)
                rvec = ri[sl]
                cvec = ci[sl]
                dotv = jnp.where(rvec == cvec, 0.0, dvec)
                wv[sl] = evv[sl] * jnp.exp(dotv * scale)
                return 0

            lax.fori_loop(0, CHUNK // 16, group, 0)
            pltpu.sync_copy(wv, w_h.at[pl.ds(off, CHUNK)])
            pltpu.sync_copy(wv, nshared.at[ri], add=True)
            return 0

        lax.fori_loop(0, NCHUNKS, chunk, 0)
        plsc.subcore_barrier()
        pltpu.sync_copy(nshared.at[pl.ds(sid * RPT, RPT)],
                        np_h.at[cid, pl.ds(sid * RPT, RPT)])

    return k(xn, rowi, coli, ev, scalin)


def _sc_spmm(out2, rowi, coli, w):
    @functools.partial(
        pl.kernel,
        out_type=jax.ShapeDtypeStruct((NC, NPAD, DH), jnp.float32),
        mesh=plsc.VectorSubcoreMesh(**_MESH),
        compiler_params=pltpu.CompilerParams(
            needs_layout_passes=False, use_tc_tiling_on_sc=False),
        scratch_types=[
            pltpu.VMEM((CHD,), jnp.int32),
            pltpu.VMEM((CHD,), jnp.int32),
            pltpu.VMEM((EPS,), jnp.int32),
            pltpu.VMEM((EPS,), jnp.float32),
            pltpu.VMEM((CHD, DH), jnp.float32),
            pltpu.VMEM((CHD, DH), jnp.float32),
            pltpu.VMEM_SHARED((NPAD, DH), jnp.float32),
            pltpu.SemaphoreType.DMA,
            pltpu.SemaphoreType.DMA,
        ],
    )
    def k(out_h, row_h, col_h, w_h, ap_h,
          ri0, ri1, ciA, wvA, g0, g1, acc, gs0, gs1):
        cid = lax.axis_index("c")
        sid = lax.axis_index("s")
        base = sid * EPS
        RI = (ri0, ri1)
        GB = (g0, g1)
        GS = (gs0, gs1)

        pltpu.sync_copy(col_h.at[pl.ds(base, EPS)], ciA)
        pltpu.sync_copy(w_h.at[pl.ds(base, EPS)], wvA)

        def cix(gg, _):
            sl = pl.ds(gg * 16, 16)
            ciA[sl] = ciA[sl] * 2 + cid
            return 0

        lax.fori_loop(0, EPS // 16, cix, 0)

        def zb(i, _):
            for sreg in range(DH // 16):
                g1[i, pl.ds(sreg * 16, 16)] = jnp.zeros((16,), jnp.float32)
            return 0

        lax.fori_loop(0, CHD, zb, 0)
        r0 = sid * RPT
        pltpu.sync_copy(g1.at[pl.ds(0, CHD)], acc.at[pl.ds(r0, CHD)])
        pltpu.sync_copy(g1.at[pl.ds(0, RPT - CHD)],
                        acc.at[pl.ds(r0 + CHD, RPT - CHD)])
        plsc.subcore_barrier()

        def fetch(gc, b):
            pltpu.sync_copy(row_h.at[pl.ds(base + gc * CHD, CHD)], RI[b])
            pltpu.make_async_copy(
                out_h.at[ciA.at[pl.ds(gc * CHD, CHD)]], GB[b],
                GS[b]).start()

        def body(gc, b):
            nb = 1 - b
            gnext = jnp.minimum(gc + 1, NCHD - 1)
            fetch(gnext, nb)
            pltpu.make_async_copy(
                out_h.at[ciA.at[pl.ds(gc * CHD, CHD)]], GB[b],
                GS[b]).wait()

            def group(gg, _):
                wgrp = wvA[pl.ds(gc * CHD + gg * 16, 16)]
                for k in range(16):
                    e = gg * 16 + k
                    we = wgrp[k]
                    for sreg in range(DH // 16):
                        s2 = pl.ds(sreg * 16, 16)
                        GB[b][e, s2] = GB[b][e, s2] * we
                return 0

            lax.fori_loop(0, CHD // 16, group, 0)
            pltpu.sync_copy(GB[b], acc.at[RI[b]], add=True)

        fetch(0, 0)

        def outer(t, _):
            body(2 * t, 0)
            body(2 * t + 1, 1)
            return 0

        lax.fori_loop(0, NCHD // 2, outer, 0)
        body(NCHD - 1, 0)
        pltpu.make_async_copy(
            out_h.at[ciA.at[pl.ds((NCHD - 1) * CHD, CHD)]], g1,
            gs1).wait()
        plsc.subcore_barrier()
        pltpu.sync_copy(acc.at[pl.ds(r0, RPT)], ap_h.at[cid, pl.ds(r0, RPT)])

    return k(out2, rowi, coli, w)


def _tc_update(x, out, ap, npart, alpha2, beta2):
    BLK = 1280

    def body(x_ref, o_ref, ap_ref, np_ref, a_ref, b_ref, on_ref):
        ae = jnp.exp(a_ref[0, 0])
        be = jnp.exp(b_ref[0, 0])
        xv = x_ref[...]
        ov = o_ref[...]
        apm = ap_ref[...]
        apv = jnp.concatenate([apm[0], apm[1]], axis=1)
        nv = jnp.sum(np_ref[...], axis=0)
        denom = be + nv[:, None] * ae + ae
        on_ref[...] = (xv * be + (apv + ov) * ae) / denom

    return pl.pallas_call(
        body,
        grid=(NPAD // BLK,),
        in_specs=[
            pl.BlockSpec((BLK, D), lambda i: (i, 0)),
            pl.BlockSpec((BLK, D), lambda i: (i, 0)),
            pl.BlockSpec((NC, BLK, DH), lambda i: (0, i, 0)),
            pl.BlockSpec((NC, BLK), lambda i: (0, i)),
            pl.BlockSpec((1, 1), lambda i: (0, 0)),
            pl.BlockSpec((1, 1), lambda i: (0, 0)),
        ],
        out_specs=pl.BlockSpec((BLK, D), lambda i: (i, 0)),
        out_shape=jax.ShapeDtypeStruct((N, D), jnp.float32),
    )(x, out, ap, npart, alpha2, beta2)


def kernel(x, edge_index, edge_vals, alpha, beta, sigma):
    ei = edge_index.astype(jnp.int32)
    rowi = ei[0]
    coli = ei[1]
    scalin = jnp.concatenate(
        [alpha.astype(jnp.float32), beta.astype(jnp.float32),
         sigma.astype(jnp.float32), jnp.zeros((13,), jnp.float32)])
    alpha2 = alpha.astype(jnp.float32).reshape(1, 1)
    beta2 = beta.astype(jnp.float32).reshape(1, 1)

    xn = _tc_normalize(x)
    w, npart = _sc_edge_weights(xn, rowi, coli, edge_vals, scalin)
    zpad_i = jnp.zeros((EPAD - E,), jnp.int32)
    rowp = jnp.concatenate([rowi, zpad_i])
    colp = jnp.concatenate([coli, zpad_i])
    wp = jnp.concatenate([w, jnp.zeros((EPAD - E,), jnp.float32)])
    out = x
    for _ in range(NUM_ITERS):
        ap = _sc_spmm(out.reshape(NC * N, DH), rowp, colp, wp)
        out = _tc_update(x, out, ap, npart, alpha2, beta2)
    return out


# speedup vs baseline: 1.9905x; 1.9905x over previous
import functools

import jax
import jax.numpy as jnp
from jax import lax
from jax.experimental import pallas as pl
from jax.experimental.pallas import tpu as pltpu
from jax.experimental.pallas import tpu_sc as plsc

N = 10000
D = 128
E = 320000
NUM_ITERS = 5

NC = 2
NS = 16
NW = NC * NS
EPW = E // NW
CHUNK = 400
NCHUNKS = EPW // CHUNK
NPAD = 10240
RPT = NPAD // NS
DH = D // NC
EPS = E // NS
CHD = 400
NCHD = EPS // CHD

_MESH = dict(core_axis_name="c", subcore_axis_name="s")


def _tc_normalize(x):
    def body(x_ref, xn_ref):
        xv = x_ref[...]
        sq = jnp.sum(xv * xv, axis=1, keepdims=True)
        xn_ref[...] = xv * lax.rsqrt(jnp.maximum(sq, 1e-12))

    return pl.pallas_call(
        body, out_shape=jax.ShapeDtypeStruct((N, D), jnp.float32)
    )(x)


def _sc_edge_weights(xn, rowi, coli, ev, scalin):
    @functools.partial(
        pl.kernel,
        out_type=(
            jax.ShapeDtypeStruct((E,), jnp.float32),
            jax.ShapeDtypeStruct((NC, NPAD), jnp.float32),
        ),
        mesh=plsc.VectorSubcoreMesh(**_MESH),
        compiler_params=pltpu.CompilerParams(needs_layout_passes=False),
        scratch_types=[
            pltpu.VMEM((CHUNK,), jnp.int32),
            pltpu.VMEM((CHUNK,), jnp.int32),
            pltpu.VMEM((CHUNK,), jnp.float32),
            pltpu.VMEM((CHUNK,), jnp.float32),
            pltpu.VMEM((CHUNK, D), jnp.float32),
            pltpu.VMEM((CHUNK, D), jnp.float32),
            pltpu.VMEM((RPT,), jnp.float32),
            pltpu.VMEM((16,), jnp.float32),
            pltpu.VMEM_SHARED((NPAD,), jnp.float32),
            pltpu.SemaphoreType.DMA,
            pltpu.SemaphoreType.DMA,
        ],
    )
    def k(xn_h, row_h, col_h, ev_h, scal_h, w_h, np_h,
          ri, ci, evv, wv, bufA, bufB, zbuf, scal_v, nshared, sem, sem2):
        cid = lax.axis_index("c")
        sid = lax.axis_index("s")
        wid = sid * NC + cid
        base = wid * EPW

        def z16(g, _):
            zbuf[pl.ds(g * 16, 16)] = jnp.zeros((16,), jnp.float32)
            return 0

        lax.fori_loop(0, RPT // 16, z16, 0)
        pltpu.sync_copy(zbuf, nshared.at[pl.ds(sid * RPT, RPT)])
        plsc.subcore_barrier()

        pltpu.sync_copy(scal_h, scal_v)
        sv = jnp.exp(scal_v[...] * -2.0)
        scale = 0.25 * sv[2]

        def chunk(g, _):
            off = base + g * CHUNK
            pltpu.sync_copy(row_h.at[pl.ds(off, CHUNK)], ri)
            pltpu.sync_copy(col_h.at[pl.ds(off, CHUNK)], ci)
            pltpu.sync_copy(ev_h.at[pl.ds(off, CHUNK)], evv)
            da = pltpu.async_copy(xn_h.at[ri], bufA, sem)
            db = pltpu.async_copy(xn_h.at[ci], bufB, sem2)
            da.wait()
            db.wait()

            lane = lax.iota(jnp.int32, 16)

            def group(gg, _):
                sl = pl.ds(gg * 16, 16)
                dvec = jnp.zeros((16,), jnp.float32)
                for k in range(16):
                    e = gg * 16 + k
                    p = [bufA[e, pl.ds(s * 16, 16)] * bufB[e, pl.ds(s * 16, 16)]
                         for s in range(8)]
                    acc = (((p[0] + p[1]) + (p[2] + p[3]))
                           + ((p[4] + p[5]) + (p[6] + p[7])))
                    dsum = jnp.sum(acc)
                    dvec = jnp.where(lane == k, dsum, dvec)
                rvec = ri[sl]
                cvec = ci[sl]
                dotv = jnp.where(rvec == cvec, 0.0, dvec)
                wv[sl] = evv[sl] * jnp.exp(dotv * scale)
                return 0

            lax.fori_loop(0, CHUNK // 16, group, 0)
            pltpu.sync_copy(wv, w_h.at[pl.ds(off, CHUNK)])
            pltpu.sync_copy(wv, nshared.at[ri], add=True)
            return 0

        lax.fori_loop(0, NCHUNKS, chunk, 0)
        plsc.subcore_barrier()
        pltpu.sync_copy(nshared.at[pl.ds(sid * RPT, RPT)],
                        np_h.at[cid, pl.ds(sid * RPT, RPT)])

    return k(xn, rowi, coli, ev, scalin)


def _sc_spmm(out2, rowi, coli, w):
    @functools.partial(
        pl.kernel,
        out_type=jax.ShapeDtypeStruct((NC, NPAD, DH), jnp.float32),
        mesh=plsc.VectorSubcoreMesh(**_MESH),
        compiler_params=pltpu.CompilerParams(
            needs_layout_passes=False, use_tc_tiling_on_sc=False),
        scratch_types=[
            pltpu.VMEM((NCHD, CHD), jnp.int32),
            pltpu.VMEM((EPS,), jnp.int32),
            pltpu.VMEM((EPS,), jnp.float32),
            pltpu.VMEM((CHD, DH), jnp.float32),
            pltpu.VMEM_SHARED((NPAD, DH), jnp.float32),
            pltpu.SemaphoreType.DMA,
        ],
    )
    def k(out_h, row_h, col_h, w_h, ap_h, riA, ciA, wvA, buf, acc, gs):
        cid = lax.axis_index("c")
        sid = lax.axis_index("s")
        base = sid * EPS

        pltpu.sync_copy(row_h.at[sid], riA)
        pltpu.sync_copy(col_h.at[pl.ds(base, EPS)], ciA)
        pltpu.sync_copy(w_h.at[pl.ds(base, EPS)], wvA)

        def cix(gg, _):
            sl = pl.ds(gg * 16, 16)
            ciA[sl] = ciA[sl] * 2 + cid
            return 0

        lax.fori_loop(0, EPS // 16, cix, 0)

        def zb(i, _):
            for sreg in range(DH // 16):
                buf[i, pl.ds(sreg * 16, 16)] = jnp.zeros((16,), jnp.float32)
            return 0

        lax.fori_loop(0, CHD, zb, 0)
        r0 = sid * RPT
        pltpu.sync_copy(buf.at[pl.ds(0, CHD)], acc.at[pl.ds(r0, CHD)])
        pltpu.sync_copy(buf.at[pl.ds(0, RPT - CHD)],
                        acc.at[pl.ds(r0 + CHD, RPT - CHD)])
        plsc.subcore_barrier()

        def chunk(gc, _):
            pltpu.async_copy(
                out_h.at[ciA.at[pl.ds(gc * CHD, CHD)]], buf, gs).wait()

            def group(gg, _):
                wgrp = wvA[pl.ds(gc * CHD + gg * 16, 16)]
                for k in range(16):
                    e = gg * 16 + k
                    we = wgrp[k]
                    for sreg in range(DH // 16):
                        s2 = pl.ds(sreg * 16, 16)
                        buf[e, s2] = buf[e, s2] * we
                return 0

            lax.fori_loop(0, CHD // 16, group, 0)
            pltpu.sync_copy(buf, acc.at[riA.at[gc]], add=True)
            return 0

        lax.fori_loop(0, NCHD, chunk, 0)
        plsc.subcore_barrier()
        pltpu.sync_copy(acc.at[pl.ds(r0, RPT)], ap_h.at[cid, pl.ds(r0, RPT)])

    return k(out2, rowi, coli, w)


def _tc_update(x, out, ap, npart, alpha2, beta2):
    BLK = 1280

    def body(x_ref, o_ref, ap_ref, np_ref, a_ref, b_ref, on_ref):
        ae = jnp.exp(a_ref[0, 0])
        be = jnp.exp(b_ref[0, 0])
        xv = x_ref[...]
        ov = o_ref[...]
        apm = ap_ref[...]
        apv = jnp.concatenate([apm[0], apm[1]], axis=1)
        nv = jnp.sum(np_ref[...], axis=0)
        denom = be + nv[:, None] * ae + ae
        on_ref[...] = (xv * be + (apv + ov) * ae) / denom

    return pl.pallas_call(
        body,
        grid=(NPAD // BLK,),
        in_specs=[
            pl.BlockSpec((BLK, D), lambda i: (i, 0)),
            pl.BlockSpec((BLK, D), lambda i: (i, 0)),
            pl.BlockSpec((NC, BLK, DH), lambda i: (0, i, 0)),
            pl.BlockSpec((NC, BLK), lambda i: (0, i)),
            pl.BlockSpec((1, 1), lambda i: (0, 0)),
            pl.BlockSpec((1, 1), lambda i: (0, 0)),
        ],
        out_specs=pl.BlockSpec((BLK, D), lambda i: (i, 0)),
        out_shape=jax.ShapeDtypeStruct((N, D), jnp.float32),
    )(x, out, ap, npart, alpha2, beta2)


def kernel(x, edge_index, edge_vals, alpha, beta, sigma):
    ei = edge_index.astype(jnp.int32)
    rowi = ei[0]
    coli = ei[1]
    scalin = jnp.concatenate(
        [alpha.astype(jnp.float32), beta.astype(jnp.float32),
         sigma.astype(jnp.float32), jnp.zeros((13,), jnp.float32)])
    alpha2 = alpha.astype(jnp.float32).reshape(1, 1)
    beta2 = beta.astype(jnp.float32).reshape(1, 1)

    xn = _tc_normalize(x)
    w, npart = _sc_edge_weights(xn, rowi, coli, edge_vals, scalin)
    rowp = rowi.reshape(NS, NCHD, CHD)
    out = x
    for _ in range(NUM_ITERS):
        ap = _sc_spmm(out.reshape(NC * N, DH), rowp, coli, w)
        out = _tc_update(x, out, ap, npart, alpha2, beta2)
    return out


# speedup vs baseline: 2.4056x; 1.2085x over previous
import functools

import jax
import jax.numpy as jnp
from jax import lax
from jax.experimental import pallas as pl
from jax.experimental.pallas import tpu as pltpu
from jax.experimental.pallas import tpu_sc as plsc

N = 10000
D = 128
E = 320000
NUM_ITERS = 5

NC = 2
NS = 16
NW = NC * NS
EPW = E // NW
CHUNK = 400
NCHUNKS = EPW // CHUNK
NPAD = 10240
RPT = NPAD // NS
DH = D // NC
EPS = E // NS
CHD = 400
NCHD = EPS // CHD

_MESH = dict(core_axis_name="c", subcore_axis_name="s")


def _tc_normalize(x):
    def body(x_ref, xn_ref):
        xv = x_ref[...]
        sq = jnp.sum(xv * xv, axis=1, keepdims=True)
        xn_ref[...] = (xv * lax.rsqrt(jnp.maximum(sq, 1e-12))
                       ).astype(jnp.bfloat16)

    return pl.pallas_call(
        body, out_shape=jax.ShapeDtypeStruct((N, D), jnp.bfloat16)
    )(x)


def _sc_edge_weights(xnb, rowb, coli, ev, scalin):
    @functools.partial(
        pl.kernel,
        out_type=(
            jax.ShapeDtypeStruct((E,), jnp.float32),
            jax.ShapeDtypeStruct((NC, NPAD), jnp.float32),
        ),
        mesh=plsc.VectorSubcoreMesh(**_MESH),
        compiler_params=pltpu.CompilerParams(
            needs_layout_passes=False, use_tc_tiling_on_sc=False),
        scratch_types=[
            pltpu.VMEM((NCHUNKS, CHUNK), jnp.int32),
            pltpu.VMEM((EPW,), jnp.int32),
            pltpu.VMEM((EPW,), jnp.float32),
            pltpu.VMEM((CHUNK,), jnp.float32),
            pltpu.VMEM((CHUNK, D), jnp.bfloat16),
            pltpu.VMEM((CHUNK, D), jnp.bfloat16),
            pltpu.VMEM((RPT,), jnp.float32),
            pltpu.VMEM((16,), jnp.float32),
            pltpu.VMEM_SHARED((NPAD,), jnp.float32),
            pltpu.SemaphoreType.DMA,
            pltpu.SemaphoreType.DMA,
        ],
    )
    def k(xn_h, row_h, col_h, ev_h, scal_h, w_h, np_h,
          riA, ciA, evA, wv, bufA, bufB, zbuf, scal_v, nshared, sem, sem2):
        cid = lax.axis_index("c")
        sid = lax.axis_index("s")
        wid = sid * NC + cid
        base = wid * EPW

        pltpu.sync_copy(row_h.at[wid], riA)
        pltpu.sync_copy(col_h.at[pl.ds(base, EPW)], ciA)
        pltpu.sync_copy(ev_h.at[pl.ds(base, EPW)], evA)

        def z16(g, _):
            zbuf[pl.ds(g * 16, 16)] = jnp.zeros((16,), jnp.float32)
            return 0

        lax.fori_loop(0, RPT // 16, z16, 0)
        pltpu.sync_copy(zbuf, nshared.at[pl.ds(sid * RPT, RPT)])
        plsc.subcore_barrier()

        pltpu.sync_copy(scal_h, scal_v)
        sv = jnp.exp(scal_v[...] * -2.0)
        scale = 0.25 * sv[2]

        def chunk(g, _):
            da = pltpu.async_copy(xn_h.at[riA.at[g]], bufA, sem)
            db = pltpu.async_copy(
                xn_h.at[ciA.at[pl.ds(g * CHUNK, CHUNK)]], bufB, sem2)
            da.wait()
            db.wait()

            lane = lax.iota(jnp.int32, 16)

            def group(gg, _):
                sl = pl.ds(gg * 16, 16)
                dvec = jnp.zeros((16,), jnp.float32)
                for k in range(16):
                    e = gg * 16 + k
                    acc0 = None
                    acc1 = None
                    for q in range(4):
                        a32 = bufA[e, pl.ds(q * 32, 32)]
                        b32 = bufB[e, pl.ds(q * 32, 32)]
                        a0, a1 = plsc.unpack(
                            a32, format=plsc.PackFormat.INTERLEAVED,
                            preferred_element_type=jnp.float32)
                        b0, b1 = plsc.unpack(
                            b32, format=plsc.PackFormat.INTERLEAVED,
                            preferred_element_type=jnp.float32)
                        if q == 0:
                            acc0 = a0 * b0
                            acc1 = a1 * b1
                        else:
                            acc0 = acc0 + a0 * b0
                            acc1 = acc1 + a1 * b1
                    dsum = jnp.sum(acc0 + acc1)
                    dvec = jnp.where(lane == k, dsum, dvec)
                rvec = riA[g, sl]
                cvec = ciA[pl.ds(g * CHUNK + gg * 16, 16)]
                dotv = jnp.where(rvec == cvec, 0.0, dvec)
                wv[sl] = evA[pl.ds(g * CHUNK + gg * 16, 16)] * jnp.exp(
                    dotv * scale)
                return 0

            lax.fori_loop(0, CHUNK // 16, group, 0)
            pltpu.sync_copy(wv, w_h.at[pl.ds(base + g * CHUNK, CHUNK)])
            pltpu.sync_copy(wv, nshared.at[riA.at[g]], add=True)
            return 0

        lax.fori_loop(0, NCHUNKS, chunk, 0)
        plsc.subcore_barrier()
        pltpu.sync_copy(nshared.at[pl.ds(sid * RPT, RPT)],
                        np_h.at[cid, pl.ds(sid * RPT, RPT)])

    return k(xnb, rowb, coli, ev, scalin)


def _sc_spmm(out2, rowi, coli, w):
    @functools.partial(
        pl.kernel,
        out_type=jax.ShapeDtypeStruct((NC, NPAD, DH), jnp.float32),
        mesh=plsc.VectorSubcoreMesh(**_MESH),
        compiler_params=pltpu.CompilerParams(
            needs_layout_passes=False, use_tc_tiling_on_sc=False),
        scratch_types=[
            pltpu.VMEM((NCHD, CHD), jnp.int32),
            pltpu.VMEM((EPS,), jnp.int32),
            pltpu.VMEM((EPS,), jnp.float32),
            pltpu.VMEM((CHD, DH), jnp.float32),
            pltpu.VMEM_SHARED((NPAD, DH), jnp.float32),
            pltpu.SemaphoreType.DMA,
        ],
    )
    def k(out_h, row_h, col_h, w_h, ap_h, riA, ciA, wvA, buf, acc, gs):
        cid = lax.axis_index("c")
        sid = lax.axis_index("s")
        base = sid * EPS

        pltpu.sync_copy(row_h.at[sid], riA)
        pltpu.sync_copy(col_h.at[pl.ds(base, EPS)], ciA)
        pltpu.sync_copy(w_h.at[pl.ds(base, EPS)], wvA)

        def cix(gg, _):
            sl = pl.ds(gg * 16, 16)
            ciA[sl] = ciA[sl] * 2 + cid
            return 0

        lax.fori_loop(0, EPS // 16, cix, 0)

        def zb(i, _):
            for sreg in range(DH // 16):
                buf[i, pl.ds(sreg * 16, 16)] = jnp.zeros((16,), jnp.float32)
            return 0

        lax.fori_loop(0, CHD, zb, 0)
        r0 = sid * RPT
        pltpu.sync_copy(buf.at[pl.ds(0, CHD)], acc.at[pl.ds(r0, CHD)])
        pltpu.sync_copy(buf.at[pl.ds(0, RPT - CHD)],
                        acc.at[pl.ds(r0 + CHD, RPT - CHD)])
        plsc.subcore_barrier()

        def chunk(gc, _):
            pltpu.async_copy(
                out_h.at[ciA.at[pl.ds(gc * CHD, CHD)]], buf, gs).wait()

            def group(gg, _):
                wgrp = wvA[pl.ds(gc * CHD + gg * 16, 16)]
                for k in range(16):
                    e = gg * 16 + k
                    we = wgrp[k]
                    for sreg in range(DH // 16):
                        s2 = pl.ds(sreg * 16, 16)
                        buf[e, s2] = buf[e, s2] * we
                return 0

            lax.fori_loop(0, CHD // 16, group, 0)
            pltpu.sync_copy(buf, acc.at[riA.at[gc]], add=True)
            return 0

        lax.fori_loop(0, NCHD, chunk, 0)
        plsc.subcore_barrier()
        pltpu.sync_copy(acc.at[pl.ds(r0, RPT)], ap_h.at[cid, pl.ds(r0, RPT)])

    return k(out2, rowi, coli, w)


def _tc_update(x, out, ap, npart, alpha2, beta2):
    BLK = 1280

    def body(x_ref, o_ref, ap_ref, np_ref, a_ref, b_ref, on_ref):
        ae = jnp.exp(a_ref[0, 0])
        be = jnp.exp(b_ref[0, 0])
        xv = x_ref[...]
        ov = o_ref[...]
        apm = ap_ref[...]
        apv = jnp.concatenate([apm[0], apm[1]], axis=1)
        nv = jnp.sum(np_ref[...], axis=0)
        denom = be + nv[:, None] * ae + ae
        on_ref[...] = (xv * be + (apv + ov) * ae) / denom

    return pl.pallas_call(
        body,
        grid=(NPAD // BLK,),
        in_specs=[
            pl.BlockSpec((BLK, D), lambda i: (i, 0)),
            pl.BlockSpec((BLK, D), lambda i: (i, 0)),
            pl.BlockSpec((NC, BLK, DH), lambda i: (0, i, 0)),
            pl.BlockSpec((NC, BLK), lambda i: (0, i)),
            pl.BlockSpec((1, 1), lambda i: (0, 0)),
            pl.BlockSpec((1, 1), lambda i: (0, 0)),
        ],
        out_specs=pl.BlockSpec((BLK, D), lambda i: (i, 0)),
        out_shape=jax.ShapeDtypeStruct((N, D), jnp.float32),
    )(x, out, ap, npart, alpha2, beta2)


def kernel(x, edge_index, edge_vals, alpha, beta, sigma):
    ei = edge_index.astype(jnp.int32)
    rowi = ei[0]
    coli = ei[1]
    scalin = jnp.concatenate(
        [alpha.astype(jnp.float32), beta.astype(jnp.float32),
         sigma.astype(jnp.float32), jnp.zeros((13,), jnp.float32)])
    alpha2 = alpha.astype(jnp.float32).reshape(1, 1)
    beta2 = beta.astype(jnp.float32).reshape(1, 1)

    xnb = _tc_normalize(x)
    rowb = rowi.reshape(NW, NCHUNKS, CHUNK)
    w, npart = _sc_edge_weights(xnb, rowb, coli, edge_vals, scalin)
    rowp = rowi.reshape(NS, NCHD, CHD)
    out = x
    for _ in range(NUM_ITERS):
        ap = _sc_spmm(out.reshape(NC * N, DH), rowp, coli, w)
        out = _tc_update(x, out, ap, npart, alpha2, beta2)
    return out
